# Initial kernel scaffold; baseline (speedup 1.0000x reference)
import jax
import jax.numpy as jnp
from jax.experimental import pallas as pl


def kernel(x, edge_index, num_prop, num_category, des_tensor, tweet_tensor, Wd, bd, Wtw, btw, Wt, bt, Wn, bn, Wc, bc, Wi, bi, Wl, bl, Wr, Wo1, bo1, Wo2, bo2):
    raise NotImplementedError("write your pallas kernel here")


# speedup vs baseline: 5.5346x; 5.5346x over previous
import jax
import jax.numpy as jnp
from jax import lax
from jax.experimental import pallas as pl
from jax.experimental.pallas import tpu as pltpu
from jax.experimental.pallas import tpu_sc as plsc

N = 10000
E = 320000
LM = 768
H = 160
HH = H // 2
P = 32

NC = 2
NS = 16
EPS = E // NS
CB = 80
NCH = EPS // CB
NP = 10240
RPS = NP // NS

RB = 1000
GRID = N // RB


def _leaky(v):
    return jnp.where(v > 0, v, 0.01 * v)


def _dot(a, b):
    return lax.dot_general(a, b, (((1,), (0,)), ((), ())),
                           preferred_element_type=jnp.float32)


def _front_body(x_r, des_r, tw_r, np_r, nc_r,
                WtT_r, WdT_r, WtwT_r, WnT_r, WcT_r, WiT_r,
                bt_r, bd_r, btw_r, bn_r, bc_r, bi_r, h_r):
    t = _leaky(_dot(x_r[...], WtT_r[...]) + bt_r[...])
    d = _leaky(_dot(des_r[...], WdT_r[...]) + bd_r[...])
    tw = _leaky(_dot(tw_r[...], WtwT_r[...]) + btw_r[...])
    n = _leaky(_dot(np_r[...], WnT_r[...]) + bn_r[...])
    c = _leaky(_dot(nc_r[...], WcT_r[...]) + bc_r[...])
    WiT = WiT_r[...]
    acc = (_dot(n, WiT[0:P]) + _dot(c, WiT[P:2 * P]) + _dot(d, WiT[2 * P:3 * P])
           + _dot(tw, WiT[3 * P:4 * P]) + _dot(t, WiT[4 * P:5 * P]) + bi_r[...])
    h = _leaky(acc)
    h_r[0] = h[:, 0:HH]
    h_r[1] = h[:, HH:H]


def _front(x, des, tweet, nprop, ncat, WtT, WdT, WtwT, WnT, WcT, WiT,
           bt2, bd2, btw2, bn2, bc2, bi2):
    row = lambda w: pl.BlockSpec((RB, w), lambda i: (i, 0))
    full = lambda s: pl.BlockSpec(s, lambda i: tuple(0 for _ in s))
    return pl.pallas_call(
        _front_body,
        grid=(GRID,),
        in_specs=[row(LM), row(LM), row(LM), row(6), row(11),
                  full((LM, P)), full((LM, P)), full((LM, P)),
                  full((6, P)), full((11, P)), full((H, H)),
                  full((1, P)), full((1, P)), full((1, P)),
                  full((1, P)), full((1, P)), full((1, H))],
        out_specs=pl.BlockSpec((NC, RB, HH), lambda i: (0, i, 0)),
        out_shape=jax.ShapeDtypeStruct((NC, N, HH), jnp.float32),
    )(x, des, tweet, nprop, ncat, WtT, WdT, WtwT, WnT, WcT, WiT,
      bt2, bd2, btw2, bn2, bc2, bi2)


def _sc_body_deg(h_hbm, srcr, dstr, z80, z1, aggp, degp,
                 src_v, dst_v, gbuf, deg_v, sem, acc):
    c = lax.axis_index("c")
    s = lax.axis_index("s")
    pltpu.sync_copy(srcr.at[c, s], src_v)
    pltpu.sync_copy(dstr.at[s], dst_v)
    o = pl.multiple_of(s * RPS, 8)
    pltpu.sync_copy(z80.at[pl.ds(o, RPS)], acc.at[pl.ds(o, RPS)])
    pltpu.sync_copy(z1, deg_v)
    plsc.subcore_barrier()
    ones16 = jnp.ones((16,), jnp.float32)

    def chunk(j, carry):
        pltpu.async_copy(h_hbm.at[src_v.at[j]], gbuf, sem).wait()
        pltpu.sync_copy(gbuf, acc.at[dst_v.at[j]], add=True)

        @pl.when(c == 0)
        def _():
            for k in range(CB // 16):
                idx = dst_v[j, pl.ds(k * 16, 16)]
                plsc.addupdate_scatter(deg_v, [idx], ones16)

        return carry

    lax.fori_loop(0, NCH, chunk, 0)
    plsc.subcore_barrier()
    pltpu.sync_copy(acc.at[pl.ds(o, RPS)], aggp.at[c, pl.ds(o, RPS)])

    @pl.when(c == 0)
    def _():
        pltpu.sync_copy(deg_v, degp.at[s])


def _sc_body(h_hbm, srcr, dstr, z80, aggp, src_v, dst_v, gbuf, sem, acc):
    c = lax.axis_index("c")
    s = lax.axis_index("s")
    pltpu.sync_copy(srcr.at[c, s], src_v)
    pltpu.sync_copy(dstr.at[s], dst_v)
    o = pl.multiple_of(s * RPS, 8)
    pltpu.sync_copy(z80.at[pl.ds(o, RPS)], acc.at[pl.ds(o, RPS)])
    plsc.subcore_barrier()

    def chunk(j, carry):
        pltpu.async_copy(h_hbm.at[src_v.at[j]], gbuf, sem).wait()
        pltpu.sync_copy(gbuf, acc.at[dst_v.at[j]], add=True)
        return carry

    lax.fori_loop(0, NCH, chunk, 0)
    plsc.subcore_barrier()
    pltpu.sync_copy(acc.at[pl.ds(o, RPS)], aggp.at[c, pl.ds(o, RPS)])


_SC_MESH = plsc.VectorSubcoreMesh(core_axis_name="c", subcore_axis_name="s")
_SC_PARAMS = pltpu.CompilerParams(use_tc_tiling_on_sc=False,
                                  needs_layout_passes=False)

_agg_deg = pl.kernel(
    _sc_body_deg,
    compiler_params=_SC_PARAMS,
    out_type=(jax.ShapeDtypeStruct((NC, NP, HH), jnp.float32),
              jax.ShapeDtypeStruct((NS, N), jnp.float32)),
    mesh=_SC_MESH,
    scratch_types=[
        pltpu.VMEM((NCH, CB), jnp.int32),
        pltpu.VMEM((NCH, CB), jnp.int32),
        pltpu.VMEM((CB, HH), jnp.float32),
        pltpu.VMEM((N,), jnp.float32),
        pltpu.SemaphoreType.DMA,
        pltpu.VMEM_SHARED((NP, HH), jnp.float32),
    ],
)

_agg = pl.kernel(
    _sc_body,
    compiler_params=_SC_PARAMS,
    out_type=jax.ShapeDtypeStruct((NC, NP, HH), jnp.float32),
    mesh=_SC_MESH,
    scratch_types=[
        pltpu.VMEM((NCH, CB), jnp.int32),
        pltpu.VMEM((NCH, CB), jnp.int32),
        pltpu.VMEM((CB, HH), jnp.float32),
        pltpu.SemaphoreType.DMA,
        pltpu.VMEM_SHARED((NP, HH), jnp.float32),
    ],
)


def _mid_body(aggp_r, degpT_r, hp_r, WlT_r, WrT_r, bl_r, h1_r, degi_r):
    deg = jnp.sum(degpT_r[...], axis=1, keepdims=True)
    degi = 1.0 / jnp.maximum(deg, 1.0)
    degi_r[...] = degi
    WlT = WlT_r[...]
    WrT = WrT_r[...]
    h1 = (_dot(aggp_r[0] * degi, WlT[0:HH]) + _dot(aggp_r[1] * degi, WlT[HH:H])
          + bl_r[...]
          + _dot(hp_r[0], WrT[0:HH]) + _dot(hp_r[1], WrT[HH:H]))
    h1_r[0] = h1[:, 0:HH]
    h1_r[1] = h1[:, HH:H]


def _mid(aggp, degpT, hp, WlT, WrT, bl2):
    row = lambda w: pl.BlockSpec((RB, w), lambda i: (i, 0))
    part = lambda w: pl.BlockSpec((NC, RB, w), lambda i: (0, i, 0))
    full = lambda s: pl.BlockSpec(s, lambda i: tuple(0 for _ in s))
    return pl.pallas_call(
        _mid_body,
        grid=(GRID,),
        in_specs=[part(HH), row(NS), part(HH),
                  full((H, H)), full((H, H)), full((1, H))],
        out_specs=[part(HH), row(1)],
        out_shape=[jax.ShapeDtypeStruct((NC, N, HH), jnp.float32),
                   jax.ShapeDtypeStruct((N, 1), jnp.float32)],
    )(aggp, degpT, hp, WlT, WrT, bl2)


def _out_body(aggp_r, degi_r, h1p_r, WlT_r, WrT_r, bl_r,
              Wo1T_r, bo1_r, Wo2T_r, bo2_r, out_r, em_r):
    degi = degi_r[...]
    WlT = WlT_r[...]
    WrT = WrT_r[...]
    h2 = (_dot(aggp_r[0] * degi, WlT[0:HH]) + _dot(aggp_r[1] * degi, WlT[HH:H])
          + bl_r[...]
          + _dot(h1p_r[0], WrT[0:HH]) + _dot(h1p_r[1], WrT[HH:H]))
    em = _leaky(_dot(h2, Wo1T_r[...]) + bo1_r[...])
    em_r[...] = em
    out_r[...] = _dot(em, Wo2T_r[...]) + bo2_r[...]


def _out(aggp, degi, h1p, WlT, WrT, bl2, Wo1T, bo12, Wo2T, bo22):
    row = lambda w: pl.BlockSpec((RB, w), lambda i: (i, 0))
    part = lambda w: pl.BlockSpec((NC, RB, w), lambda i: (0, i, 0))
    full = lambda s: pl.BlockSpec(s, lambda i: tuple(0 for _ in s))
    return pl.pallas_call(
        _out_body,
        grid=(GRID,),
        in_specs=[part(HH), row(1), part(HH),
                  full((H, H)), full((H, H)), full((1, H)),
                  full((H, H)), full((1, H)), full((H, 2)), full((1, 2))],
        out_specs=[row(2), row(H)],
        out_shape=[jax.ShapeDtypeStruct((N, 2), jnp.float32),
                   jax.ShapeDtypeStruct((N, H), jnp.float32)],
    )(aggp, degi, h1p, WlT, WrT, bl2, Wo1T, bo12, Wo2T, bo22)


def kernel(x, edge_index, num_prop, num_category, des_tensor, tweet_tensor,
           Wd, bd, Wtw, btw, Wt, bt, Wn, bn, Wc, bc, Wi, bi,
           Wl, bl, Wr, Wo1, bo1, Wo2, bo2):
    src = edge_index[0]
    srcr = jnp.stack([src, src + N]).reshape(NC, NS, NCH, CB)
    dstr = edge_index[1].reshape(NS, NCH, CB)
    z80 = jnp.zeros((NP, HH), jnp.float32)
    z1 = jnp.zeros((N,), jnp.float32)

    r2 = lambda v: v.reshape(1, -1)
    hp = _front(x, des_tensor, tweet_tensor, num_prop, num_category,
                Wt.T, Wd.T, Wtw.T, Wn.T, Wc.T, Wi.T,
                r2(bt), r2(bd), r2(btw), r2(bn), r2(bc), r2(bi))
    h2n = hp.reshape(NC * N, HH)

    aggp, degp = _agg_deg(h2n, srcr, dstr, z80, z1)
    h1p, degi = _mid(aggp, degp.T, hp, Wl.T, Wr.T, r2(bl))
    aggp2 = _agg(h1p.reshape(NC * N, HH), srcr, dstr, z80)
    out, em = _out(aggp2, degi, h1p, Wl.T, Wr.T, r2(bl),
                   Wo1.T, r2(bo1), Wo2.T, r2(bo2))
    return (out, em)
